# baseline (speedup 1.0000x reference)
import functools

import jax
import jax.numpy as jnp
from jax import lax
from jax.experimental import pallas as pl
from jax.experimental.pallas import tpu as pltpu
from jax.experimental.pallas import tpu_sc as plsc

N = 10000
E = 320000
D = 128
H = 128
OUT = 8
NGRAPHS = 64

NC = 2
NS = 16
NW = NC * NS
LANES = 16

NP = 10240
HNP = NP // 2
EPAD = 327680
PK = 14
PKM = (1 << PK) - 1
ES = 2048
NSLAB = EPAD // ES
SRT = 320
SPAN = 2 * SRT
LCAP = 16384
LRCAP = 8192
CHB = 64


@functools.lru_cache(maxsize=None)
def _make_seg_sum_v3():
    Dp = H
    mesh = plsc.VectorSubcoreMesh(core_axis_name="c", subcore_axis_name="s",
                                  num_cores=NC, num_subcores=NS)
    @functools.partial(
        pl.kernel,
        out_type=jax.ShapeDtypeStruct((NP, Dp), jnp.float32),
        mesh=mesh,
        compiler_params=pltpu.CompilerParams(needs_layout_passes=False),
        scratch_types=[
            pltpu.VMEM((SRT + 1, Dp), jnp.float32),
            pltpu.VMEM((ES,), jnp.int32),
            pltpu.VMEM((LCAP + 80,), jnp.int32),
            pltpu.VMEM((LRCAP + CHB + 16,), jnp.int32),
            pltpu.VMEM((2, CHB, Dp), jnp.float32),
            pltpu.VMEM((2, CHB), jnp.int32),
            pltpu.VMEM_SHARED((HNP, Dp), jnp.float32),
            pltpu.SemaphoreType.DMA((2,)),
        ],
    )
    def seg(h_hbm, ep_hbm, out_hbm, hloc, eslab, llist, llr, stage2,
            dstb2, acc, sem2):
        c = lax.axis_index("c")
        s = lax.axis_index("s")
        lo_base = s * SPAN
        clo = c * HNP
        zero16 = jnp.zeros((LANES,), jnp.float32)
        iota16 = lax.iota(jnp.int32, LANES)

        def zrow(i, _):
            for q in range(Dp // LANES):
                stage2[0, i, pl.ds(q * LANES, LANES)] = zero16
            return _
        lax.fori_loop(0, CHB, zrow, None)
        for k in range(HNP // NS // CHB):
            pltpu.sync_copy(stage2.at[0],
                            acc.at[pl.ds(s * (HNP // NS) + k * CHB, CHB)])
        for q in range(Dp // LANES):
            hloc[SRT, pl.ds(q * LANES, LANES)] = zero16

        def slab_body(b, ptr):
            pltpu.sync_copy(ep_hbm.at[pl.ds(b * ES, ES)], eslab)

            def scan_body(i, ptr):
                e = eslab[pl.ds(i * LANES, LANES)]
                srcv = lax.shift_right_logical(e, PK)
                dstv = lax.bitwise_and(e, PKM)
                off = srcv - lo_base
                m = ((dstv >= clo) & (dstv < clo + HNP)
                     & (off >= 0) & (off < SPAN))
                pos = plsc.cumsum(m.astype(jnp.int32))
                plsc.store_scatter(llist, [ptr + pos - 1], e, mask=m)
                return jnp.minimum(ptr + pos[LANES - 1], LCAP)
            return lax.fori_loop(0, ES // LANES, scan_body, ptr)
        cnt_all = lax.fori_loop(0, NSLAB, slab_body, jnp.int32(0))

        plsc.subcore_barrier()

        nit = (cnt_all + LANES - 1) // LANES
        for r in range(2):
            lo_r = lo_base + r * SRT
            pltpu.sync_copy(h_hbm.at[pl.ds(lo_r, SRT)],
                            hloc.at[pl.ds(0, SRT)])

            def rc_body(i, pr):
                e = llist[pl.ds(i * LANES, LANES)]
                srcv = lax.shift_right_logical(e, PK)
                off = srcv - lo_r
                valid = (i * LANES + iota16) < cnt_all
                m = valid & (off >= 0) & (off < SRT)
                pos = plsc.cumsum(m.astype(jnp.int32))
                plsc.store_scatter(llr, [pr + pos - 1], e, mask=m)
                return jnp.minimum(pr + pos[LANES - 1], LRCAP)
            pr = lax.fori_loop(0, nit, rc_body, jnp.int32(0))

            e_pad = jnp.full((16,), ((lo_r + SRT) << PK), jnp.int32) + clo
            for k in range(CHB // LANES):
                llr[pl.ds(pr + k * LANES, LANES)] = e_pad

            nch = (pr + CHB - 1) // CHB

            def chunk(ch, _):
                p = lax.rem(ch, 2)
                @pl.when(ch >= 2)
                def _wait():
                    pltpu.make_async_copy(stage2.at[p],
                                          acc.at[dstb2.at[p]],
                                          sem2.at[p]).wait()
                base = ch * CHB
                for g in range(CHB // LANES):
                    e = llr[pl.ds(base + g * LANES, LANES)]
                    srcloc = lax.shift_right_logical(e, PK) - lo_r
                    dstloc = lax.bitwise_and(e, PKM) - clo
                    dstb2[p, pl.ds(g * LANES, LANES)] = dstloc
                    for l in range(LANES):
                        sl = srcloc[l]
                        for q in range(Dp // LANES):
                            stage2[p, g * LANES + l,
                                   pl.ds(q * LANES, LANES)] = (
                                hloc[sl, pl.ds(q * LANES, LANES)])

                pltpu.make_async_copy(stage2.at[p], acc.at[dstb2.at[p]],
                                      sem2.at[p]).start(add=True)
                return _
            lax.fori_loop(0, nch, chunk, None)

            @pl.when(nch >= 1)
            def _d1():
                p = lax.rem(nch - 1, 2)
                pltpu.make_async_copy(stage2.at[p], acc.at[dstb2.at[p]],
                                      sem2.at[p]).wait()

            @pl.when(nch >= 2)
            def _d2():
                p = lax.rem(nch, 2)
                pltpu.make_async_copy(stage2.at[p], acc.at[dstb2.at[p]],
                                      sem2.at[p]).wait()

        plsc.subcore_barrier()
        pltpu.sync_copy(acc.at[pl.ds(s * (HNP // NS), HNP // NS)],
                        out_hbm.at[pl.ds(clo + s * (HNP // NS), HNP // NS)])

    return seg


def _seg_sum(h, ep):
    return _make_seg_sum_v3()(h, ep)


def _embed_body(x_ref, ge_ref, he_ref, W1a_ref, W1b_ref, o_ref):
    n = lax.broadcasted_iota(jnp.int32, (NP, 1), 0)
    gid = jnp.where(n < 16, n // 2, 0)
    onehot_g = (gid == lax.broadcasted_iota(jnp.int32, (NP, 8), 1)
                ).astype(jnp.float32)
    hemi = n % 2
    onehot_h = (hemi == lax.broadcasted_iota(jnp.int32, (NP, 2), 1)
                ).astype(jnp.float32)
    emb_w = jnp.concatenate([
        jnp.dot(ge_ref[...], W1b_ref[0:2, :],
                preferred_element_type=jnp.float32,
                precision=lax.Precision.HIGHEST),
        jnp.dot(he_ref[...], W1b_ref[2:4, :],
                preferred_element_type=jnp.float32,
                precision=lax.Precision.HIGHEST),
    ], axis=0)
    onehot = jnp.concatenate([onehot_g, onehot_h], axis=1)
    u = (jnp.dot(x_ref[...], W1a_ref[...],
                 preferred_element_type=jnp.float32,
                 precision=lax.Precision.HIGHEST) +
         jnp.dot(onehot, emb_w, preferred_element_type=jnp.float32,
                 precision=lax.Precision.HIGHEST))
    mask = (n < N).astype(jnp.float32)
    o_ref[...] = u * mask


def _embed(x_p, group_emb, hemi_emb, W1a, W1b):
    return pl.pallas_call(
        _embed_body,
        out_shape=jax.ShapeDtypeStruct((NP, H), jnp.float32),
    )(x_p, group_emb, hemi_emb, W1a, W1b)


def _bn_relu_mm(y, gamma, beta, W2, b2, mask):
    y = y * mask
    mu = jnp.sum(y, axis=0, keepdims=True) / N
    var = jnp.sum(y * y, axis=0, keepdims=True) / N - mu * mu
    y = gamma * (y - mu) / jnp.sqrt(var + 1e-5) + beta
    y = jnp.maximum(y, 0.0) * mask
    o = jnp.dot(y, W2, preferred_element_type=jnp.float32,
                precision=lax.Precision.HIGHEST) + b2
    return jnp.maximum(o, 0.0) * mask


def _layer_math(h, agg, W1, b1, gamma, beta, W2, b2):
    mask = (lax.broadcasted_iota(jnp.int32, (NP, 1), 0) < N).astype(
        jnp.float32)
    z = h + agg
    y = jnp.dot(z, W1, preferred_element_type=jnp.float32,
                precision=lax.Precision.HIGHEST) + b1
    return _bn_relu_mm(y, gamma, beta, W2, b2, mask)


def _dense0_body(u_ref, a_ref, b1_ref, g_ref, be_ref, W2_ref, b2_ref, o_ref):
    mask = (lax.broadcasted_iota(jnp.int32, (NP, 1), 0) < N).astype(
        jnp.float32)
    y = u_ref[...] + a_ref[...] + b1_ref[...]
    o_ref[...] = _bn_relu_mm(y, g_ref[...], be_ref[...], W2_ref[...],
                             b2_ref[...], mask)


def _dense0(u, agg, b1, gamma, beta, W2, b2):
    return pl.pallas_call(
        _dense0_body,
        out_shape=jax.ShapeDtypeStruct((NP, H), jnp.float32),
    )(u, agg, b1.reshape(1, H), gamma.reshape(1, H), beta.reshape(1, H),
      W2, b2.reshape(1, H))


def _dense_body(h_ref, a_ref, W1_ref, b1_ref, g_ref, be_ref, W2_ref, b2_ref,
                o_ref):
    o_ref[...] = _layer_math(h_ref[...], a_ref[...], W1_ref[...],
                             b1_ref[...], g_ref[...], be_ref[...],
                             W2_ref[...], b2_ref[...])


def _dense(h, agg, W1, b1, gamma, beta, W2, b2):
    return pl.pallas_call(
        _dense_body,
        out_shape=jax.ShapeDtypeStruct((NP, H), jnp.float32),
    )(h, agg, W1, b1.reshape(1, H), gamma.reshape(1, H), beta.reshape(1, H),
      W2, b2.reshape(1, H))


def _final_body(h_ref, a_ref, W1_ref, b1_ref, g_ref, be_ref, W2_ref, b2_ref,
                batch_ref, Wm1_ref, bm1_ref, Wm2_ref, bm2_ref, o_ref):
    h3 = _layer_math(h_ref[...], a_ref[...], W1_ref[...],
                     b1_ref[...], g_ref[...], be_ref[...], W2_ref[...],
                     b2_ref[...])
    onehot = (batch_ref[...] ==
              lax.broadcasted_iota(jnp.int32, (NP, NGRAPHS), 1)
              ).astype(jnp.float32)
    pooled = lax.dot_general(onehot, h3, (((0,), (0,)), ((), ())),
                             preferred_element_type=jnp.float32,
                             precision=lax.Precision.HIGHEST)
    y = jnp.maximum(
        jnp.dot(pooled, Wm1_ref[...], preferred_element_type=jnp.float32,
                precision=lax.Precision.HIGHEST) + bm1_ref[...], 0.0)
    o_ref[...] = jnp.dot(y, Wm2_ref[...], preferred_element_type=jnp.float32,
                         precision=lax.Precision.HIGHEST) + bm2_ref[...]


def _final(h, agg, W1, b1, gamma, beta, W2, b2, batch_p, Wm1, bm1, Wm2, bm2):
    return pl.pallas_call(
        _final_body,
        out_shape=jax.ShapeDtypeStruct((NGRAPHS, OUT), jnp.float32),
    )(h, agg, W1, b1.reshape(1, H), gamma.reshape(1, H), beta.reshape(1, H),
      W2, b2.reshape(1, H), batch_p, Wm1, bm1.reshape(1, H), Wm2,
      bm2.reshape(1, OUT))


def kernel(x, edge_index, edge_attr, batch, group_emb, hemi_emb,
           W1_0, b1_0, gamma_0, beta_0, W2_0, b2_0,
           W1_1, b1_1, gamma_1, beta_1, W2_1, b2_1,
           W1_2, b1_2, gamma_2, beta_2, W2_2, b2_2,
           Wm1, bm1, Wm2, bm2):
    src = edge_index[0]
    dst = edge_index[1]
    pad_e = EPAD - E
    ep = jnp.concatenate([
        jnp.bitwise_or(jnp.left_shift(src, PK), dst),
        jnp.full((pad_e,), PKM, jnp.int32),
    ])
    x_p = jnp.pad(x, ((0, NP - N), (0, 0)))
    batch_p = jnp.pad(batch, (0, NP - N),
                      constant_values=NGRAPHS).reshape(NP, 1)

    u0 = _embed(x_p, group_emb, hemi_emb, W1_0[:D], W1_0[D:])
    agg0 = _seg_sum(u0, ep)
    h1 = _dense0(u0, agg0, b1_0, gamma_0, beta_0, W2_0, b2_0)
    agg1 = _seg_sum(h1, ep)
    h2 = _dense(h1, agg1, W1_1, b1_1, gamma_1, beta_1, W2_1, b2_1)
    agg2 = _seg_sum(h2, ep)
    return _final(h2, agg2, W1_2, b1_2, gamma_2, beta_2, W2_2, b2_2,
                  batch_p, Wm1, bm1, Wm2, bm2)


# speedup vs baseline: 1.1684x; 1.1684x over previous
import functools

import jax
import jax.numpy as jnp
from jax import lax
from jax.experimental import pallas as pl
from jax.experimental.pallas import tpu as pltpu
from jax.experimental.pallas import tpu_sc as plsc

N = 10000
E = 320000
D = 128
H = 128
OUT = 8
NGRAPHS = 64

NC = 2
NS = 16
NW = NC * NS
LANES = 16

NP = 10240
HNP = NP // 2
EPAD = 327680
PK = 14
PKM = (1 << PK) - 1
ES = 2048
NSLAB = EPAD // ES
SRT = 320
SPAN = 2 * SRT
LCAP = 16384
LRCAP = 8192
CHB = 64


@functools.lru_cache(maxsize=None)
def _make_seg_sum_v3():
    Dp = H
    mesh = plsc.VectorSubcoreMesh(core_axis_name="c", subcore_axis_name="s",
                                  num_cores=NC, num_subcores=NS)
    @functools.partial(
        pl.kernel,
        out_type=jax.ShapeDtypeStruct((NP, Dp), jnp.float32),
        mesh=mesh,
        compiler_params=pltpu.CompilerParams(needs_layout_passes=False),
        scratch_types=[
            pltpu.VMEM((SRT + 1, Dp), jnp.float32),
            pltpu.VMEM((2, ES), jnp.int32),
            pltpu.VMEM((LCAP + 80,), jnp.int32),
            pltpu.VMEM((LRCAP + CHB + 16,), jnp.int32),
            pltpu.VMEM((2, CHB, Dp), jnp.float32),
            pltpu.VMEM((2, CHB), jnp.int32),
            pltpu.VMEM_SHARED((HNP, Dp), jnp.float32),
            pltpu.SemaphoreType.DMA((2,)),
            pltpu.SemaphoreType.DMA((2,)),
        ],
    )
    def seg(h_hbm, ep_hbm, out_hbm, hloc, eslab, llist, llr, stage2,
            dstb2, acc, sem2, sem_es):
        c = lax.axis_index("c")
        s = lax.axis_index("s")
        lo_base = s * SPAN
        clo = c * HNP
        zero16 = jnp.zeros((LANES,), jnp.float32)
        iota16 = lax.iota(jnp.int32, LANES)

        def zrow(i, _):
            for q in range(Dp // LANES):
                stage2[0, i, pl.ds(q * LANES, LANES)] = zero16
            return _
        lax.fori_loop(0, CHB, zrow, None)
        for k in range(HNP // NS // CHB):
            pltpu.sync_copy(stage2.at[0],
                            acc.at[pl.ds(s * (HNP // NS) + k * CHB, CHB)])
        for q in range(Dp // LANES):
            hloc[SRT, pl.ds(q * LANES, LANES)] = zero16

        pltpu.async_copy(ep_hbm.at[pl.ds(0, ES)], eslab.at[0], sem_es.at[0])

        def slab_body(b, ptr):
            p = lax.rem(b, 2)
            pltpu.make_async_copy(ep_hbm.at[pl.ds(b * ES, ES)],
                                  eslab.at[p], sem_es.at[p]).wait()

            @pl.when(b + 1 < NSLAB)
            def _prefetch():
                pn = lax.rem(b + 1, 2)
                pltpu.async_copy(ep_hbm.at[pl.ds((b + 1) * ES, ES)],
                                 eslab.at[pn], sem_es.at[pn])

            def scan_body(i, ptr):
                e = eslab[p, pl.ds(i * LANES, LANES)]
                srcv = lax.shift_right_logical(e, PK)
                dstv = lax.bitwise_and(e, PKM)
                off = srcv - lo_base
                m = ((dstv >= clo) & (dstv < clo + HNP)
                     & (off >= 0) & (off < SPAN))
                pos = plsc.cumsum(m.astype(jnp.int32))
                plsc.store_scatter(llist, [ptr + pos - 1], e, mask=m)
                cnt = plsc.all_reduce_population_count(m)[0]
                return jnp.minimum(ptr + cnt, LCAP)
            return lax.fori_loop(0, ES // LANES, scan_body, ptr)
        cnt_all = lax.fori_loop(0, NSLAB, slab_body, jnp.int32(0))

        plsc.subcore_barrier()

        nit = (cnt_all + LANES - 1) // LANES
        for r in range(2):
            lo_r = lo_base + r * SRT
            pltpu.sync_copy(h_hbm.at[pl.ds(lo_r, SRT)],
                            hloc.at[pl.ds(0, SRT)])

            def rc_body(i, pr):
                e = llist[pl.ds(i * LANES, LANES)]
                srcv = lax.shift_right_logical(e, PK)
                off = srcv - lo_r
                valid = (i * LANES + iota16) < cnt_all
                m = valid & (off >= 0) & (off < SRT)
                pos = plsc.cumsum(m.astype(jnp.int32))
                plsc.store_scatter(llr, [pr + pos - 1], e, mask=m)
                cnt = plsc.all_reduce_population_count(m)[0]
                return jnp.minimum(pr + cnt, LRCAP)
            pr = lax.fori_loop(0, nit, rc_body, jnp.int32(0))

            e_pad = jnp.full((16,), ((lo_r + SRT) << PK), jnp.int32) + clo
            for k in range(CHB // LANES):
                llr[pl.ds(pr + k * LANES, LANES)] = e_pad

            nch = (pr + CHB - 1) // CHB

            def chunk(ch, _):
                p = lax.rem(ch, 2)
                @pl.when(ch >= 2)
                def _wait():
                    pltpu.make_async_copy(stage2.at[p],
                                          acc.at[dstb2.at[p]],
                                          sem2.at[p]).wait()
                base = ch * CHB
                for g in range(CHB // LANES):
                    e = llr[pl.ds(base + g * LANES, LANES)]
                    srcloc = lax.shift_right_logical(e, PK) - lo_r
                    dstloc = lax.bitwise_and(e, PKM) - clo
                    dstb2[p, pl.ds(g * LANES, LANES)] = dstloc
                    for l in range(LANES):
                        sl = srcloc[l]
                        for q in range(Dp // LANES):
                            stage2[p, g * LANES + l,
                                   pl.ds(q * LANES, LANES)] = (
                                hloc[sl, pl.ds(q * LANES, LANES)])

                pltpu.make_async_copy(stage2.at[p], acc.at[dstb2.at[p]],
                                      sem2.at[p]).start(add=True)
                return _
            lax.fori_loop(0, nch, chunk, None)

            @pl.when(nch >= 1)
            def _d1():
                p = lax.rem(nch - 1, 2)
                pltpu.make_async_copy(stage2.at[p], acc.at[dstb2.at[p]],
                                      sem2.at[p]).wait()

            @pl.when(nch >= 2)
            def _d2():
                p = lax.rem(nch, 2)
                pltpu.make_async_copy(stage2.at[p], acc.at[dstb2.at[p]],
                                      sem2.at[p]).wait()

        plsc.subcore_barrier()
        pltpu.sync_copy(acc.at[pl.ds(s * (HNP // NS), HNP // NS)],
                        out_hbm.at[pl.ds(clo + s * (HNP // NS), HNP // NS)])

    return seg


def _seg_sum(h, ep):
    return _make_seg_sum_v3()(h, ep)


def _embed_body(x_ref, ge_ref, he_ref, W1a_ref, W1b_ref, o_ref):
    n = lax.broadcasted_iota(jnp.int32, (NP, 1), 0)
    gid = jnp.where(n < 16, n // 2, 0)
    onehot_g = (gid == lax.broadcasted_iota(jnp.int32, (NP, 8), 1)
                ).astype(jnp.float32)
    hemi = n % 2
    onehot_h = (hemi == lax.broadcasted_iota(jnp.int32, (NP, 2), 1)
                ).astype(jnp.float32)
    emb_w = jnp.concatenate([
        jnp.dot(ge_ref[...], W1b_ref[0:2, :],
                preferred_element_type=jnp.float32,
                precision=lax.Precision.HIGHEST),
        jnp.dot(he_ref[...], W1b_ref[2:4, :],
                preferred_element_type=jnp.float32,
                precision=lax.Precision.HIGHEST),
    ], axis=0)
    onehot = jnp.concatenate([onehot_g, onehot_h], axis=1)
    u = (jnp.dot(x_ref[...], W1a_ref[...],
                 preferred_element_type=jnp.float32,
                 precision=lax.Precision.HIGHEST) +
         jnp.dot(onehot, emb_w, preferred_element_type=jnp.float32,
                 precision=lax.Precision.HIGHEST))
    mask = (n < N).astype(jnp.float32)
    o_ref[...] = u * mask


def _embed(x_p, group_emb, hemi_emb, W1a, W1b):
    return pl.pallas_call(
        _embed_body,
        out_shape=jax.ShapeDtypeStruct((NP, H), jnp.float32),
    )(x_p, group_emb, hemi_emb, W1a, W1b)


def _bn_relu_mm(y, gamma, beta, W2, b2, mask):
    y = y * mask
    mu = jnp.sum(y, axis=0, keepdims=True) / N
    var = jnp.sum(y * y, axis=0, keepdims=True) / N - mu * mu
    y = gamma * (y - mu) / jnp.sqrt(var + 1e-5) + beta
    y = jnp.maximum(y, 0.0) * mask
    o = jnp.dot(y, W2, preferred_element_type=jnp.float32,
                precision=lax.Precision.HIGHEST) + b2
    return jnp.maximum(o, 0.0) * mask


def _layer_math(h, agg, W1, b1, gamma, beta, W2, b2):
    mask = (lax.broadcasted_iota(jnp.int32, (NP, 1), 0) < N).astype(
        jnp.float32)
    z = h + agg
    y = jnp.dot(z, W1, preferred_element_type=jnp.float32,
                precision=lax.Precision.HIGHEST) + b1
    return _bn_relu_mm(y, gamma, beta, W2, b2, mask)


def _dense0_body(u_ref, a_ref, b1_ref, g_ref, be_ref, W2_ref, b2_ref, o_ref):
    mask = (lax.broadcasted_iota(jnp.int32, (NP, 1), 0) < N).astype(
        jnp.float32)
    y = u_ref[...] + a_ref[...] + b1_ref[...]
    o_ref[...] = _bn_relu_mm(y, g_ref[...], be_ref[...], W2_ref[...],
                             b2_ref[...], mask)


def _dense0(u, agg, b1, gamma, beta, W2, b2):
    return pl.pallas_call(
        _dense0_body,
        out_shape=jax.ShapeDtypeStruct((NP, H), jnp.float32),
    )(u, agg, b1.reshape(1, H), gamma.reshape(1, H), beta.reshape(1, H),
      W2, b2.reshape(1, H))


def _dense_body(h_ref, a_ref, W1_ref, b1_ref, g_ref, be_ref, W2_ref, b2_ref,
                o_ref):
    o_ref[...] = _layer_math(h_ref[...], a_ref[...], W1_ref[...],
                             b1_ref[...], g_ref[...], be_ref[...],
                             W2_ref[...], b2_ref[...])


def _dense(h, agg, W1, b1, gamma, beta, W2, b2):
    return pl.pallas_call(
        _dense_body,
        out_shape=jax.ShapeDtypeStruct((NP, H), jnp.float32),
    )(h, agg, W1, b1.reshape(1, H), gamma.reshape(1, H), beta.reshape(1, H),
      W2, b2.reshape(1, H))


def _final_body(h_ref, a_ref, W1_ref, b1_ref, g_ref, be_ref, W2_ref, b2_ref,
                batch_ref, Wm1_ref, bm1_ref, Wm2_ref, bm2_ref, o_ref):
    h3 = _layer_math(h_ref[...], a_ref[...], W1_ref[...],
                     b1_ref[...], g_ref[...], be_ref[...], W2_ref[...],
                     b2_ref[...])
    onehot = (batch_ref[...] ==
              lax.broadcasted_iota(jnp.int32, (NP, NGRAPHS), 1)
              ).astype(jnp.float32)
    pooled = lax.dot_general(onehot, h3, (((0,), (0,)), ((), ())),
                             preferred_element_type=jnp.float32,
                             precision=lax.Precision.HIGHEST)
    y = jnp.maximum(
        jnp.dot(pooled, Wm1_ref[...], preferred_element_type=jnp.float32,
                precision=lax.Precision.HIGHEST) + bm1_ref[...], 0.0)
    o_ref[...] = jnp.dot(y, Wm2_ref[...], preferred_element_type=jnp.float32,
                         precision=lax.Precision.HIGHEST) + bm2_ref[...]


def _final(h, agg, W1, b1, gamma, beta, W2, b2, batch_p, Wm1, bm1, Wm2, bm2):
    return pl.pallas_call(
        _final_body,
        out_shape=jax.ShapeDtypeStruct((NGRAPHS, OUT), jnp.float32),
    )(h, agg, W1, b1.reshape(1, H), gamma.reshape(1, H), beta.reshape(1, H),
      W2, b2.reshape(1, H), batch_p, Wm1, bm1.reshape(1, H), Wm2,
      bm2.reshape(1, OUT))


def kernel(x, edge_index, edge_attr, batch, group_emb, hemi_emb,
           W1_0, b1_0, gamma_0, beta_0, W2_0, b2_0,
           W1_1, b1_1, gamma_1, beta_1, W2_1, b2_1,
           W1_2, b1_2, gamma_2, beta_2, W2_2, b2_2,
           Wm1, bm1, Wm2, bm2):
    src = edge_index[0]
    dst = edge_index[1]
    pad_e = EPAD - E
    ep = jnp.concatenate([
        jnp.bitwise_or(jnp.left_shift(src, PK), dst),
        jnp.full((pad_e,), PKM, jnp.int32),
    ])
    x_p = jnp.pad(x, ((0, NP - N), (0, 0)))
    batch_p = jnp.pad(batch, (0, NP - N),
                      constant_values=NGRAPHS).reshape(NP, 1)

    u0 = _embed(x_p, group_emb, hemi_emb, W1_0[:D], W1_0[D:])
    agg0 = _seg_sum(u0, ep)
    h1 = _dense0(u0, agg0, b1_0, gamma_0, beta_0, W2_0, b2_0)
    agg1 = _seg_sum(h1, ep)
    h2 = _dense(h1, agg1, W1_1, b1_1, gamma_1, beta_1, W2_1, b2_1)
    agg2 = _seg_sum(h2, ep)
    return _final(h2, agg2, W1_2, b1_2, gamma_2, beta_2, W2_2, b2_2,
                  batch_p, Wm1, bm1, Wm2, bm2)


# speedup vs baseline: 1.5732x; 1.3464x over previous
import functools

import jax
import jax.numpy as jnp
from jax import lax
from jax.experimental import pallas as pl
from jax.experimental.pallas import tpu as pltpu
from jax.experimental.pallas import tpu_sc as plsc

N = 10000
E = 320000
D = 128
H = 128
OUT = 8
NGRAPHS = 64

NC = 2
NS = 16
NW = NC * NS
LANES = 16

NP = 10240
HNP = NP // 2
EPAD = 327680
PK = 14
PKM = (1 << PK) - 1
ES = 2048
NSLAB = EPAD // ES
SRT = 320
SPAN = 2 * SRT
LCAP = 16384
LRCAP = 8192
CHB = 64


@functools.lru_cache(maxsize=None)
def _make_seg_sum_v3():
    Dp = H
    mesh = plsc.VectorSubcoreMesh(core_axis_name="c", subcore_axis_name="s",
                                  num_cores=NC, num_subcores=NS)
    @functools.partial(
        pl.kernel,
        out_type=jax.ShapeDtypeStruct((NP, Dp), jnp.float32),
        mesh=mesh,
        compiler_params=pltpu.CompilerParams(needs_layout_passes=False),
        scratch_types=[
            pltpu.VMEM_SHARED((NS * (SRT + 8), Dp), jnp.float32),
            pltpu.VMEM((2, ES), jnp.int32),
            pltpu.VMEM((LCAP + 80,), jnp.int32),
            pltpu.VMEM((LRCAP + CHB + 16,), jnp.int32),
            pltpu.VMEM((2, CHB, Dp), jnp.float32),
            pltpu.VMEM((2, CHB), jnp.int32),
            pltpu.VMEM((2, CHB), jnp.int32),
            pltpu.VMEM_SHARED((HNP, Dp), jnp.float32),
            pltpu.SemaphoreType.DMA((2,)),
            pltpu.SemaphoreType.DMA((2,)),
            pltpu.SemaphoreType.DMA((2,)),
        ],
    )
    def seg(h_hbm, ep_hbm, out_hbm, hloc, eslab, llist, llr, stage2,
            dstb2, srcb2, acc, sem2, sem_es, semg):
        c = lax.axis_index("c")
        s = lax.axis_index("s")
        lo_base = s * SPAN
        clo = c * HNP
        zero16 = jnp.zeros((LANES,), jnp.float32)
        iota16 = lax.iota(jnp.int32, LANES)

        def zrow(i, _):
            for q in range(Dp // LANES):
                stage2[0, i, pl.ds(q * LANES, LANES)] = zero16
            return _
        lax.fori_loop(0, CHB, zrow, None)
        for k in range(HNP // NS // CHB):
            pltpu.sync_copy(stage2.at[0],
                            acc.at[pl.ds(s * (HNP // NS) + k * CHB, CHB)])
        hbase = s * (SRT + 8)
        pltpu.sync_copy(stage2.at[0, pl.ds(0, 1)],
                        hloc.at[pl.ds(hbase + SRT, 1)])

        pltpu.async_copy(ep_hbm.at[pl.ds(0, ES)], eslab.at[0], sem_es.at[0])

        def slab_body(b, ptr):
            p = lax.rem(b, 2)
            pltpu.make_async_copy(ep_hbm.at[pl.ds(b * ES, ES)],
                                  eslab.at[p], sem_es.at[p]).wait()

            @pl.when(b + 1 < NSLAB)
            def _prefetch():
                pn = lax.rem(b + 1, 2)
                pltpu.async_copy(ep_hbm.at[pl.ds((b + 1) * ES, ES)],
                                 eslab.at[pn], sem_es.at[pn])

            def scan_body(i, ptr):
                e = eslab[p, pl.ds(i * LANES, LANES)]
                srcv = lax.shift_right_logical(e, PK)
                dstv = lax.bitwise_and(e, PKM)
                off = srcv - lo_base
                m = ((dstv >= clo) & (dstv < clo + HNP)
                     & (off >= 0) & (off < SPAN))
                pos = plsc.cumsum(m.astype(jnp.int32))
                plsc.store_scatter(llist, [ptr + pos - 1], e, mask=m)
                cnt = plsc.all_reduce_population_count(m)[0]
                return jnp.minimum(ptr + cnt, LCAP)
            return lax.fori_loop(0, ES // LANES, scan_body, ptr)
        cnt_all = lax.fori_loop(0, NSLAB, slab_body, jnp.int32(0))

        plsc.subcore_barrier()

        nit = (cnt_all + LANES - 1) // LANES
        for r in range(2):
            lo_r = lo_base + r * SRT
            pltpu.sync_copy(h_hbm.at[pl.ds(lo_r, SRT)],
                            hloc.at[pl.ds(hbase, SRT)])

            def rc_body(i, pr):
                e = llist[pl.ds(i * LANES, LANES)]
                srcv = lax.shift_right_logical(e, PK)
                off = srcv - lo_r
                valid = (i * LANES + iota16) < cnt_all
                m = valid & (off >= 0) & (off < SRT)
                pos = plsc.cumsum(m.astype(jnp.int32))
                plsc.store_scatter(llr, [pr + pos - 1], e, mask=m)
                cnt = plsc.all_reduce_population_count(m)[0]
                return jnp.minimum(pr + cnt, LRCAP)
            pr = lax.fori_loop(0, nit, rc_body, jnp.int32(0))

            e_pad = jnp.full((16,), ((lo_r + SRT) << PK), jnp.int32) + clo
            for k in range(CHB // LANES):
                llr[pl.ds(pr + k * LANES, LANES)] = e_pad

            nch = (pr + CHB - 1) // CHB

            def chunk(ch, _):
                p = lax.rem(ch, 2)
                @pl.when(ch >= 2)
                def _wait():
                    pltpu.make_async_copy(stage2.at[p],
                                          acc.at[dstb2.at[p]],
                                          sem2.at[p]).wait()
                base = ch * CHB
                for g in range(CHB // LANES):
                    e = llr[pl.ds(base + g * LANES, LANES)]
                    srcb2[p, pl.ds(g * LANES, LANES)] = (
                        lax.shift_right_logical(e, PK) + (hbase - lo_r))
                    dstb2[p, pl.ds(g * LANES, LANES)] = (
                        lax.bitwise_and(e, PKM) - clo)
                pltpu.async_copy(hloc.at[srcb2.at[p]], stage2.at[p],
                                 semg.at[p])

                @pl.when(ch >= 1)
                def _prev():
                    q = 1 - p
                    pltpu.make_async_copy(hloc.at[srcb2.at[q]],
                                          stage2.at[q], semg.at[q]).wait()
                    pltpu.make_async_copy(stage2.at[q], acc.at[dstb2.at[q]],
                                          sem2.at[q]).start(add=True)
                return _
            lax.fori_loop(0, nch, chunk, None)

            @pl.when(nch >= 1)
            def _d1():
                q = lax.rem(nch - 1, 2)
                pltpu.make_async_copy(hloc.at[srcb2.at[q]], stage2.at[q],
                                      semg.at[q]).wait()
                pltpu.make_async_copy(stage2.at[q], acc.at[dstb2.at[q]],
                                      sem2.at[q]).start(add=True)
                pltpu.make_async_copy(stage2.at[q], acc.at[dstb2.at[q]],
                                      sem2.at[q]).wait()

            @pl.when(nch >= 2)
            def _d2():
                q = lax.rem(nch, 2)
                pltpu.make_async_copy(stage2.at[q], acc.at[dstb2.at[q]],
                                      sem2.at[q]).wait()

        plsc.subcore_barrier()
        pltpu.sync_copy(acc.at[pl.ds(s * (HNP // NS), HNP // NS)],
                        out_hbm.at[pl.ds(clo + s * (HNP // NS), HNP // NS)])

    return seg


def _seg_sum(h, ep):
    return _make_seg_sum_v3()(h, ep)


def _embed_body(x_ref, ge_ref, he_ref, W1a_ref, W1b_ref, o_ref):
    n = lax.broadcasted_iota(jnp.int32, (NP, 1), 0)
    gid = jnp.where(n < 16, n // 2, 0)
    onehot_g = (gid == lax.broadcasted_iota(jnp.int32, (NP, 8), 1)
                ).astype(jnp.float32)
    hemi = n % 2
    onehot_h = (hemi == lax.broadcasted_iota(jnp.int32, (NP, 2), 1)
                ).astype(jnp.float32)
    emb_w = jnp.concatenate([
        jnp.dot(ge_ref[...], W1b_ref[0:2, :],
                preferred_element_type=jnp.float32,
                precision=lax.Precision.HIGHEST),
        jnp.dot(he_ref[...], W1b_ref[2:4, :],
                preferred_element_type=jnp.float32,
                precision=lax.Precision.HIGHEST),
    ], axis=0)
    onehot = jnp.concatenate([onehot_g, onehot_h], axis=1)
    u = (jnp.dot(x_ref[...], W1a_ref[...],
                 preferred_element_type=jnp.float32,
                 precision=lax.Precision.HIGHEST) +
         jnp.dot(onehot, emb_w, preferred_element_type=jnp.float32,
                 precision=lax.Precision.HIGHEST))
    mask = (n < N).astype(jnp.float32)
    o_ref[...] = u * mask


def _embed(x_p, group_emb, hemi_emb, W1a, W1b):
    return pl.pallas_call(
        _embed_body,
        out_shape=jax.ShapeDtypeStruct((NP, H), jnp.float32),
    )(x_p, group_emb, hemi_emb, W1a, W1b)


def _bn_relu_mm(y, gamma, beta, W2, b2, mask):
    y = y * mask
    mu = jnp.sum(y, axis=0, keepdims=True) / N
    var = jnp.sum(y * y, axis=0, keepdims=True) / N - mu * mu
    y = gamma * (y - mu) / jnp.sqrt(var + 1e-5) + beta
    y = jnp.maximum(y, 0.0) * mask
    o = jnp.dot(y, W2, preferred_element_type=jnp.float32,
                precision=lax.Precision.HIGHEST) + b2
    return jnp.maximum(o, 0.0) * mask


def _layer_math(h, agg, W1, b1, gamma, beta, W2, b2):
    mask = (lax.broadcasted_iota(jnp.int32, (NP, 1), 0) < N).astype(
        jnp.float32)
    z = h + agg
    y = jnp.dot(z, W1, preferred_element_type=jnp.float32,
                precision=lax.Precision.HIGHEST) + b1
    return _bn_relu_mm(y, gamma, beta, W2, b2, mask)


def _dense0_body(u_ref, a_ref, b1_ref, g_ref, be_ref, W2_ref, b2_ref, o_ref):
    mask = (lax.broadcasted_iota(jnp.int32, (NP, 1), 0) < N).astype(
        jnp.float32)
    y = u_ref[...] + a_ref[...] + b1_ref[...]
    o_ref[...] = _bn_relu_mm(y, g_ref[...], be_ref[...], W2_ref[...],
                             b2_ref[...], mask)


def _dense0(u, agg, b1, gamma, beta, W2, b2):
    return pl.pallas_call(
        _dense0_body,
        out_shape=jax.ShapeDtypeStruct((NP, H), jnp.float32),
    )(u, agg, b1.reshape(1, H), gamma.reshape(1, H), beta.reshape(1, H),
      W2, b2.reshape(1, H))


def _dense_body(h_ref, a_ref, W1_ref, b1_ref, g_ref, be_ref, W2_ref, b2_ref,
                o_ref):
    o_ref[...] = _layer_math(h_ref[...], a_ref[...], W1_ref[...],
                             b1_ref[...], g_ref[...], be_ref[...],
                             W2_ref[...], b2_ref[...])


def _dense(h, agg, W1, b1, gamma, beta, W2, b2):
    return pl.pallas_call(
        _dense_body,
        out_shape=jax.ShapeDtypeStruct((NP, H), jnp.float32),
    )(h, agg, W1, b1.reshape(1, H), gamma.reshape(1, H), beta.reshape(1, H),
      W2, b2.reshape(1, H))


def _final_body(h_ref, a_ref, W1_ref, b1_ref, g_ref, be_ref, W2_ref, b2_ref,
                batch_ref, Wm1_ref, bm1_ref, Wm2_ref, bm2_ref, o_ref):
    h3 = _layer_math(h_ref[...], a_ref[...], W1_ref[...],
                     b1_ref[...], g_ref[...], be_ref[...], W2_ref[...],
                     b2_ref[...])
    onehot = (batch_ref[...] ==
              lax.broadcasted_iota(jnp.int32, (NP, NGRAPHS), 1)
              ).astype(jnp.float32)
    pooled = lax.dot_general(onehot, h3, (((0,), (0,)), ((), ())),
                             preferred_element_type=jnp.float32,
                             precision=lax.Precision.HIGHEST)
    y = jnp.maximum(
        jnp.dot(pooled, Wm1_ref[...], preferred_element_type=jnp.float32,
                precision=lax.Precision.HIGHEST) + bm1_ref[...], 0.0)
    o_ref[...] = jnp.dot(y, Wm2_ref[...], preferred_element_type=jnp.float32,
                         precision=lax.Precision.HIGHEST) + bm2_ref[...]


def _final(h, agg, W1, b1, gamma, beta, W2, b2, batch_p, Wm1, bm1, Wm2, bm2):
    return pl.pallas_call(
        _final_body,
        out_shape=jax.ShapeDtypeStruct((NGRAPHS, OUT), jnp.float32),
    )(h, agg, W1, b1.reshape(1, H), gamma.reshape(1, H), beta.reshape(1, H),
      W2, b2.reshape(1, H), batch_p, Wm1, bm1.reshape(1, H), Wm2,
      bm2.reshape(1, OUT))


def kernel(x, edge_index, edge_attr, batch, group_emb, hemi_emb,
           W1_0, b1_0, gamma_0, beta_0, W2_0, b2_0,
           W1_1, b1_1, gamma_1, beta_1, W2_1, b2_1,
           W1_2, b1_2, gamma_2, beta_2, W2_2, b2_2,
           Wm1, bm1, Wm2, bm2):
    src = edge_index[0]
    dst = edge_index[1]
    pad_e = EPAD - E
    ep = jnp.concatenate([
        jnp.bitwise_or(jnp.left_shift(src, PK), dst),
        jnp.full((pad_e,), PKM, jnp.int32),
    ])
    x_p = jnp.pad(x, ((0, NP - N), (0, 0)))
    batch_p = jnp.pad(batch, (0, NP - N),
                      constant_values=NGRAPHS).reshape(NP, 1)

    u0 = _embed(x_p, group_emb, hemi_emb, W1_0[:D], W1_0[D:])
    agg0 = _seg_sum(u0, ep)
    h1 = _dense0(u0, agg0, b1_0, gamma_0, beta_0, W2_0, b2_0)
    agg1 = _seg_sum(h1, ep)
    h2 = _dense(h1, agg1, W1_1, b1_1, gamma_1, beta_1, W2_1, b2_1)
    agg2 = _seg_sum(h2, ep)
    return _final(h2, agg2, W1_2, b1_2, gamma_2, beta_2, W2_2, b2_2,
                  batch_p, Wm1, bm1, Wm2, bm2)


# speedup vs baseline: 3.4968x; 2.2228x over previous
import functools

import jax
import jax.numpy as jnp
from jax import lax
from jax.experimental import pallas as pl
from jax.experimental.pallas import tpu as pltpu
from jax.experimental.pallas import tpu_sc as plsc

N = 10000
E = 320000
D = 128
H = 128
OUT = 8
NGRAPHS = 64

NC = 2
NS = 16
NW = NC * NS
LANES = 16

NP = 10240
HNP = NP // 2
EPAD = 327680
PK = 14
PKM = (1 << PK) - 1
ES = 2048
NSLAB = EPAD // ES
SRT = 320
RB = NS * SRT
NR = NP // RB
SPT = 10
LCAP = 16384
LRCAP = 8192
CHB = 64


@functools.lru_cache(maxsize=None)
def _make_seg_sum_v3():
    Dp = H
    mesh = plsc.VectorSubcoreMesh(core_axis_name="c", subcore_axis_name="s",
                                  num_cores=NC, num_subcores=NS)
    @functools.partial(
        pl.kernel,
        out_type=jax.ShapeDtypeStruct((NP, Dp), jnp.float32),
        mesh=mesh,
        compiler_params=pltpu.CompilerParams(needs_layout_passes=False),
        scratch_types=[
            pltpu.VMEM_SHARED((RB + 8, Dp), jnp.float32),
            pltpu.VMEM((2, ES), jnp.int32),
            pltpu.VMEM((LCAP + 80,), jnp.int32),
            pltpu.VMEM((LRCAP + CHB + 16,), jnp.int32),
            pltpu.VMEM((2, CHB, Dp), jnp.float32),
            pltpu.VMEM((2, CHB), jnp.int32),
            pltpu.VMEM((2, CHB), jnp.int32),
            pltpu.VMEM_SHARED((HNP, Dp), jnp.float32),
            pltpu.SemaphoreType.DMA((2,)),
            pltpu.SemaphoreType.DMA((2,)),
            pltpu.SemaphoreType.DMA((2,)),
        ],
    )
    def seg(h_hbm, ep_hbm, out_hbm, hloc, eslab, llist, llr, stage2,
            dstb2, srcb2, acc, sem2, sem_es, semg):
        c = lax.axis_index("c")
        s = lax.axis_index("s")
        clo = c * HNP
        zero16 = jnp.zeros((LANES,), jnp.float32)
        iota16 = lax.iota(jnp.int32, LANES)

        def zrow(i, _):
            for q in range(Dp // LANES):
                stage2[0, i, pl.ds(q * LANES, LANES)] = zero16
            return _
        lax.fori_loop(0, CHB, zrow, None)
        for k in range(HNP // NS // CHB):
            pltpu.sync_copy(stage2.at[0],
                            acc.at[pl.ds(s * (HNP // NS) + k * CHB, CHB)])
        @pl.when(s == 0)
        def _zsent():
            pltpu.sync_copy(stage2.at[0, pl.ds(0, 1)],
                            hloc.at[pl.ds(RB, 1)])

        slab0 = s * SPT
        pltpu.async_copy(ep_hbm.at[pl.ds(slab0 * ES, ES)], eslab.at[0],
                         sem_es.at[0])

        def slab_body(b, ptr):
            p = lax.rem(b, 2)
            pltpu.make_async_copy(ep_hbm.at[pl.ds((slab0 + b) * ES, ES)],
                                  eslab.at[p], sem_es.at[p]).wait()

            @pl.when(b + 1 < SPT)
            def _prefetch():
                pn = lax.rem(b + 1, 2)
                pltpu.async_copy(
                    ep_hbm.at[pl.ds((slab0 + b + 1) * ES, ES)],
                    eslab.at[pn], sem_es.at[pn])

            def scan_body(i, ptr):
                e = eslab[p, pl.ds(i * LANES, LANES)]
                dstv = lax.bitwise_and(e, PKM)
                m = (dstv >= clo) & (dstv < clo + HNP)
                pos = plsc.cumsum(m.astype(jnp.int32))
                plsc.store_scatter(llist, [ptr + pos - 1], e, mask=m)
                cnt = plsc.all_reduce_population_count(m)[0]
                return jnp.minimum(ptr + cnt, LCAP)
            return lax.fori_loop(0, ES // LANES, scan_body, ptr)
        cnt_all = lax.fori_loop(0, SPT, slab_body, jnp.int32(0))

        plsc.subcore_barrier()

        nit = (cnt_all + LANES - 1) // LANES
        for r in range(NR):
            lo_r = r * RB
            pltpu.sync_copy(h_hbm.at[pl.ds(lo_r + s * SRT, SRT)],
                            hloc.at[pl.ds(s * SRT, SRT)])

            def rc_body(i, pr):
                e = llist[pl.ds(i * LANES, LANES)]
                srcv = lax.shift_right_logical(e, PK)
                off = srcv - lo_r
                valid = (i * LANES + iota16) < cnt_all
                m = valid & (off >= 0) & (off < RB)
                pos = plsc.cumsum(m.astype(jnp.int32))
                plsc.store_scatter(llr, [pr + pos - 1], e, mask=m)
                cnt = plsc.all_reduce_population_count(m)[0]
                return jnp.minimum(pr + cnt, LRCAP)
            pr = lax.fori_loop(0, nit, rc_body, jnp.int32(0))

            e_pad = jnp.full((16,), ((lo_r + RB) << PK), jnp.int32) + clo
            for k in range(CHB // LANES):
                llr[pl.ds(pr + k * LANES, LANES)] = e_pad

            nch = (pr + CHB - 1) // CHB
            plsc.subcore_barrier()

            def chunk(ch, _):
                p = lax.rem(ch, 2)
                @pl.when(ch >= 2)
                def _wait():
                    pltpu.make_async_copy(stage2.at[p],
                                          acc.at[dstb2.at[p]],
                                          sem2.at[p]).wait()
                base = ch * CHB
                for g in range(CHB // LANES):
                    e = llr[pl.ds(base + g * LANES, LANES)]
                    srcb2[p, pl.ds(g * LANES, LANES)] = (
                        lax.shift_right_logical(e, PK) - lo_r)
                    dstb2[p, pl.ds(g * LANES, LANES)] = (
                        lax.bitwise_and(e, PKM) - clo)
                pltpu.async_copy(hloc.at[srcb2.at[p]], stage2.at[p],
                                 semg.at[p])

                @pl.when(ch >= 1)
                def _prev():
                    q = 1 - p
                    pltpu.make_async_copy(hloc.at[srcb2.at[q]],
                                          stage2.at[q], semg.at[q]).wait()
                    pltpu.make_async_copy(stage2.at[q], acc.at[dstb2.at[q]],
                                          sem2.at[q]).start(add=True)
                return _
            lax.fori_loop(0, nch, chunk, None)

            @pl.when(nch >= 1)
            def _d1():
                q = lax.rem(nch - 1, 2)
                pltpu.make_async_copy(hloc.at[srcb2.at[q]], stage2.at[q],
                                      semg.at[q]).wait()
                pltpu.make_async_copy(stage2.at[q], acc.at[dstb2.at[q]],
                                      sem2.at[q]).start(add=True)
                pltpu.make_async_copy(stage2.at[q], acc.at[dstb2.at[q]],
                                      sem2.at[q]).wait()

            @pl.when(nch >= 2)
            def _d2():
                q = lax.rem(nch, 2)
                pltpu.make_async_copy(stage2.at[q], acc.at[dstb2.at[q]],
                                      sem2.at[q]).wait()

            plsc.subcore_barrier()

        pltpu.sync_copy(acc.at[pl.ds(s * (HNP // NS), HNP // NS)],
                        out_hbm.at[pl.ds(clo + s * (HNP // NS), HNP // NS)])

    return seg


def _seg_sum(h, ep):
    return _make_seg_sum_v3()(h, ep)


def _embed_body(x_ref, ge_ref, he_ref, W1a_ref, W1b_ref, o_ref):
    n = lax.broadcasted_iota(jnp.int32, (NP, 1), 0)
    gid = jnp.where(n < 16, n // 2, 0)
    onehot_g = (gid == lax.broadcasted_iota(jnp.int32, (NP, 8), 1)
                ).astype(jnp.float32)
    hemi = n % 2
    onehot_h = (hemi == lax.broadcasted_iota(jnp.int32, (NP, 2), 1)
                ).astype(jnp.float32)
    emb_w = jnp.concatenate([
        jnp.dot(ge_ref[...], W1b_ref[0:2, :],
                preferred_element_type=jnp.float32,
                precision=lax.Precision.HIGHEST),
        jnp.dot(he_ref[...], W1b_ref[2:4, :],
                preferred_element_type=jnp.float32,
                precision=lax.Precision.HIGHEST),
    ], axis=0)
    onehot = jnp.concatenate([onehot_g, onehot_h], axis=1)
    u = (jnp.dot(x_ref[...], W1a_ref[...],
                 preferred_element_type=jnp.float32,
                 precision=lax.Precision.HIGHEST) +
         jnp.dot(onehot, emb_w, preferred_element_type=jnp.float32,
                 precision=lax.Precision.HIGHEST))
    mask = (n < N).astype(jnp.float32)
    o_ref[...] = u * mask


def _embed(x_p, group_emb, hemi_emb, W1a, W1b):
    return pl.pallas_call(
        _embed_body,
        out_shape=jax.ShapeDtypeStruct((NP, H), jnp.float32),
    )(x_p, group_emb, hemi_emb, W1a, W1b)


def _bn_relu_mm(y, gamma, beta, W2, b2, mask):
    y = y * mask
    mu = jnp.sum(y, axis=0, keepdims=True) / N
    var = jnp.sum(y * y, axis=0, keepdims=True) / N - mu * mu
    y = gamma * (y - mu) / jnp.sqrt(var + 1e-5) + beta
    y = jnp.maximum(y, 0.0) * mask
    o = jnp.dot(y, W2, preferred_element_type=jnp.float32,
                precision=lax.Precision.HIGHEST) + b2
    return jnp.maximum(o, 0.0) * mask


def _layer_math(h, agg, W1, b1, gamma, beta, W2, b2):
    mask = (lax.broadcasted_iota(jnp.int32, (NP, 1), 0) < N).astype(
        jnp.float32)
    z = h + agg
    y = jnp.dot(z, W1, preferred_element_type=jnp.float32,
                precision=lax.Precision.HIGHEST) + b1
    return _bn_relu_mm(y, gamma, beta, W2, b2, mask)


def _dense0_body(u_ref, a_ref, b1_ref, g_ref, be_ref, W2_ref, b2_ref, o_ref):
    mask = (lax.broadcasted_iota(jnp.int32, (NP, 1), 0) < N).astype(
        jnp.float32)
    y = u_ref[...] + a_ref[...] + b1_ref[...]
    o_ref[...] = _bn_relu_mm(y, g_ref[...], be_ref[...], W2_ref[...],
                             b2_ref[...], mask)


def _dense0(u, agg, b1, gamma, beta, W2, b2):
    return pl.pallas_call(
        _dense0_body,
        out_shape=jax.ShapeDtypeStruct((NP, H), jnp.float32),
    )(u, agg, b1.reshape(1, H), gamma.reshape(1, H), beta.reshape(1, H),
      W2, b2.reshape(1, H))


def _dense_body(h_ref, a_ref, W1_ref, b1_ref, g_ref, be_ref, W2_ref, b2_ref,
                o_ref):
    o_ref[...] = _layer_math(h_ref[...], a_ref[...], W1_ref[...],
                             b1_ref[...], g_ref[...], be_ref[...],
                             W2_ref[...], b2_ref[...])


def _dense(h, agg, W1, b1, gamma, beta, W2, b2):
    return pl.pallas_call(
        _dense_body,
        out_shape=jax.ShapeDtypeStruct((NP, H), jnp.float32),
    )(h, agg, W1, b1.reshape(1, H), gamma.reshape(1, H), beta.reshape(1, H),
      W2, b2.reshape(1, H))


def _final_body(h_ref, a_ref, W1_ref, b1_ref, g_ref, be_ref, W2_ref, b2_ref,
                batch_ref, Wm1_ref, bm1_ref, Wm2_ref, bm2_ref, o_ref):
    h3 = _layer_math(h_ref[...], a_ref[...], W1_ref[...],
                     b1_ref[...], g_ref[...], be_ref[...], W2_ref[...],
                     b2_ref[...])
    onehot = (batch_ref[...] ==
              lax.broadcasted_iota(jnp.int32, (NP, NGRAPHS), 1)
              ).astype(jnp.float32)
    pooled = lax.dot_general(onehot, h3, (((0,), (0,)), ((), ())),
                             preferred_element_type=jnp.float32,
                             precision=lax.Precision.HIGHEST)
    y = jnp.maximum(
        jnp.dot(pooled, Wm1_ref[...], preferred_element_type=jnp.float32,
                precision=lax.Precision.HIGHEST) + bm1_ref[...], 0.0)
    o_ref[...] = jnp.dot(y, Wm2_ref[...], preferred_element_type=jnp.float32,
                         precision=lax.Precision.HIGHEST) + bm2_ref[...]


def _final(h, agg, W1, b1, gamma, beta, W2, b2, batch_p, Wm1, bm1, Wm2, bm2):
    return pl.pallas_call(
        _final_body,
        out_shape=jax.ShapeDtypeStruct((NGRAPHS, OUT), jnp.float32),
    )(h, agg, W1, b1.reshape(1, H), gamma.reshape(1, H), beta.reshape(1, H),
      W2, b2.reshape(1, H), batch_p, Wm1, bm1.reshape(1, H), Wm2,
      bm2.reshape(1, OUT))


def kernel(x, edge_index, edge_attr, batch, group_emb, hemi_emb,
           W1_0, b1_0, gamma_0, beta_0, W2_0, b2_0,
           W1_1, b1_1, gamma_1, beta_1, W2_1, b2_1,
           W1_2, b1_2, gamma_2, beta_2, W2_2, b2_2,
           Wm1, bm1, Wm2, bm2):
    src = edge_index[0]
    dst = edge_index[1]
    pad_e = EPAD - E
    ep = jnp.concatenate([
        jnp.bitwise_or(jnp.left_shift(src, PK), dst),
        jnp.full((pad_e,), PKM, jnp.int32),
    ])
    x_p = jnp.pad(x, ((0, NP - N), (0, 0)))
    batch_p = jnp.pad(batch, (0, NP - N),
                      constant_values=NGRAPHS).reshape(NP, 1)

    u0 = _embed(x_p, group_emb, hemi_emb, W1_0[:D], W1_0[D:])
    agg0 = _seg_sum(u0, ep)
    h1 = _dense0(u0, agg0, b1_0, gamma_0, beta_0, W2_0, b2_0)
    agg1 = _seg_sum(h1, ep)
    h2 = _dense(h1, agg1, W1_1, b1_1, gamma_1, beta_1, W2_1, b2_1)
    agg2 = _seg_sum(h2, ep)
    return _final(h2, agg2, W1_2, b1_2, gamma_2, beta_2, W2_2, b2_2,
                  batch_p, Wm1, bm1, Wm2, bm2)


# speedup vs baseline: 3.6147x; 1.0337x over previous
import functools

import jax
import jax.numpy as jnp
from jax import lax
from jax.experimental import pallas as pl
from jax.experimental.pallas import tpu as pltpu
from jax.experimental.pallas import tpu_sc as plsc

N = 10000
E = 320000
D = 128
H = 128
OUT = 8
NGRAPHS = 64

NC = 2
NS = 16
NW = NC * NS
LANES = 16

NP = 10240
HNP = NP // 2
EPAD = 327680
PK = 14
PKM = (1 << PK) - 1
ES = 2048
NSLAB = EPAD // ES
SRT = 320
RB = NS * SRT
NR = NP // RB
SPT = 10
LCAP = 16384
LRCAP = 8192
CHB = 64


@functools.lru_cache(maxsize=None)
def _make_seg_sum_v3():
    Dp = H
    mesh = plsc.VectorSubcoreMesh(core_axis_name="c", subcore_axis_name="s",
                                  num_cores=NC, num_subcores=NS)
    @functools.partial(
        pl.kernel,
        out_type=jax.ShapeDtypeStruct((NP, Dp), jnp.float32),
        mesh=mesh,
        compiler_params=pltpu.CompilerParams(needs_layout_passes=False),
        scratch_types=[
            pltpu.VMEM_SHARED((RB + 8, Dp), jnp.float32),
            pltpu.VMEM((2, ES), jnp.int32),
            pltpu.VMEM((LCAP + 80,), jnp.int32),
            pltpu.VMEM((LRCAP + CHB + 16,), jnp.int32),
            pltpu.VMEM((2, CHB, Dp), jnp.float32),
            pltpu.VMEM((2, CHB), jnp.int32),
            pltpu.VMEM((2, CHB), jnp.int32),
            pltpu.VMEM_SHARED((HNP, Dp), jnp.float32),
            pltpu.SemaphoreType.DMA((2,)),
            pltpu.SemaphoreType.DMA((2,)),
            pltpu.SemaphoreType.DMA((2,)),
        ],
    )
    def seg(h_hbm, ep_hbm, out_hbm, hloc, eslab, llist, llr, stage2,
            dstb2, srcb2, acc, sem2, sem_es, semg):
        c = lax.axis_index("c")
        s = lax.axis_index("s")
        clo = c * HNP
        zero16 = jnp.zeros((LANES,), jnp.float32)
        iota16 = lax.iota(jnp.int32, LANES)

        def zrow(i, _):
            for q in range(Dp // LANES):
                stage2[0, i, pl.ds(q * LANES, LANES)] = zero16
            return _
        lax.fori_loop(0, CHB, zrow, None)
        for k in range(HNP // NS // CHB):
            pltpu.sync_copy(stage2.at[0],
                            acc.at[pl.ds(s * (HNP // NS) + k * CHB, CHB)])
        @pl.when(s == 0)
        def _zsent():
            pltpu.sync_copy(stage2.at[0, pl.ds(0, 1)],
                            hloc.at[pl.ds(RB, 1)])

        slab0 = s * SPT
        pltpu.async_copy(ep_hbm.at[pl.ds(slab0 * ES, ES)], eslab.at[0],
                         sem_es.at[0])

        def slab_body(b, ptr):
            p = lax.rem(b, 2)
            pltpu.make_async_copy(ep_hbm.at[pl.ds((slab0 + b) * ES, ES)],
                                  eslab.at[p], sem_es.at[p]).wait()

            @pl.when(b + 1 < SPT)
            def _prefetch():
                pn = lax.rem(b + 1, 2)
                pltpu.async_copy(
                    ep_hbm.at[pl.ds((slab0 + b + 1) * ES, ES)],
                    eslab.at[pn], sem_es.at[pn])

            def scan_body(i, ptr):
                e = eslab[p, pl.ds(i * LANES, LANES)]
                dstv = lax.bitwise_and(e, PKM)
                m = (dstv >= clo) & (dstv < clo + HNP)
                pos = plsc.cumsum(m.astype(jnp.int32))
                plsc.store_scatter(llist, [ptr + pos - 1], e, mask=m)
                cnt = plsc.all_reduce_population_count(m)[0]
                return jnp.minimum(ptr + cnt, LCAP)
            return lax.fori_loop(0, ES // LANES, scan_body, ptr)
        cnt_all = lax.fori_loop(0, SPT, slab_body, jnp.int32(0))

        plsc.subcore_barrier()

        nit = (cnt_all + LANES - 1) // LANES
        for r in range(NR):
            lo_r = r * RB
            pltpu.async_copy(h_hbm.at[pl.ds(lo_r + s * SRT, SRT)],
                             hloc.at[pl.ds(s * SRT, SRT)], semg.at[0])

            def rc_body(i, pr):
                e = llist[pl.ds(i * LANES, LANES)]
                srcv = lax.shift_right_logical(e, PK)
                off = srcv - lo_r
                valid = (i * LANES + iota16) < cnt_all
                m = valid & (off >= 0) & (off < RB)
                pos = plsc.cumsum(m.astype(jnp.int32))
                plsc.store_scatter(llr, [pr + pos - 1], e, mask=m)
                cnt = plsc.all_reduce_population_count(m)[0]
                return jnp.minimum(pr + cnt, LRCAP)
            pr = lax.fori_loop(0, nit, rc_body, jnp.int32(0))

            e_pad = jnp.full((16,), ((lo_r + RB) << PK), jnp.int32) + clo
            for k in range(CHB // LANES):
                llr[pl.ds(pr + k * LANES, LANES)] = e_pad

            nch = (pr + CHB - 1) // CHB
            pltpu.make_async_copy(h_hbm.at[pl.ds(lo_r + s * SRT, SRT)],
                                  hloc.at[pl.ds(s * SRT, SRT)],
                                  semg.at[0]).wait()
            plsc.subcore_barrier()

            def chunk(ch, _):
                p = lax.rem(ch, 2)
                @pl.when(ch >= 2)
                def _wait():
                    pltpu.make_async_copy(stage2.at[p],
                                          acc.at[dstb2.at[p]],
                                          sem2.at[p]).wait()
                base = ch * CHB
                for g in range(CHB // LANES):
                    e = llr[pl.ds(base + g * LANES, LANES)]
                    srcb2[p, pl.ds(g * LANES, LANES)] = (
                        lax.shift_right_logical(e, PK) - lo_r)
                    dstb2[p, pl.ds(g * LANES, LANES)] = (
                        lax.bitwise_and(e, PKM) - clo)
                pltpu.async_copy(hloc.at[srcb2.at[p]], stage2.at[p],
                                 semg.at[p])

                @pl.when(ch >= 1)
                def _prev():
                    q = 1 - p
                    pltpu.make_async_copy(hloc.at[srcb2.at[q]],
                                          stage2.at[q], semg.at[q]).wait()
                    pltpu.make_async_copy(stage2.at[q], acc.at[dstb2.at[q]],
                                          sem2.at[q]).start(add=True)
                return _
            lax.fori_loop(0, nch, chunk, None)

            @pl.when(nch >= 1)
            def _d1():
                q = lax.rem(nch - 1, 2)
                pltpu.make_async_copy(hloc.at[srcb2.at[q]], stage2.at[q],
                                      semg.at[q]).wait()
                pltpu.make_async_copy(stage2.at[q], acc.at[dstb2.at[q]],
                                      sem2.at[q]).start(add=True)
                pltpu.make_async_copy(stage2.at[q], acc.at[dstb2.at[q]],
                                      sem2.at[q]).wait()

            @pl.when(nch >= 2)
            def _d2():
                q = lax.rem(nch, 2)
                pltpu.make_async_copy(stage2.at[q], acc.at[dstb2.at[q]],
                                      sem2.at[q]).wait()

            plsc.subcore_barrier()

        pltpu.sync_copy(acc.at[pl.ds(s * (HNP // NS), HNP // NS)],
                        out_hbm.at[pl.ds(clo + s * (HNP // NS), HNP // NS)])

    return seg


def _seg_sum(h, ep):
    return _make_seg_sum_v3()(h, ep)


def _embed_body(x_ref, ge_ref, he_ref, W1a_ref, W1b_ref, o_ref):
    n = lax.broadcasted_iota(jnp.int32, (NP, 1), 0)
    gid = jnp.where(n < 16, n // 2, 0)
    onehot_g = (gid == lax.broadcasted_iota(jnp.int32, (NP, 8), 1)
                ).astype(jnp.float32)
    hemi = n % 2
    onehot_h = (hemi == lax.broadcasted_iota(jnp.int32, (NP, 2), 1)
                ).astype(jnp.float32)
    emb_w = jnp.concatenate([
        jnp.dot(ge_ref[...], W1b_ref[0:2, :],
                preferred_element_type=jnp.float32,
                precision=lax.Precision.HIGHEST),
        jnp.dot(he_ref[...], W1b_ref[2:4, :],
                preferred_element_type=jnp.float32,
                precision=lax.Precision.HIGHEST),
    ], axis=0)
    onehot = jnp.concatenate([onehot_g, onehot_h], axis=1)
    u = (jnp.dot(x_ref[...], W1a_ref[...],
                 preferred_element_type=jnp.float32,
                 precision=lax.Precision.HIGHEST) +
         jnp.dot(onehot, emb_w, preferred_element_type=jnp.float32,
                 precision=lax.Precision.HIGHEST))
    mask = (n < N).astype(jnp.float32)
    o_ref[...] = u * mask


def _embed(x_p, group_emb, hemi_emb, W1a, W1b):
    return pl.pallas_call(
        _embed_body,
        out_shape=jax.ShapeDtypeStruct((NP, H), jnp.float32),
    )(x_p, group_emb, hemi_emb, W1a, W1b)


def _bn_relu_mm(y, gamma, beta, W2, b2, mask):
    y = y * mask
    mu = jnp.sum(y, axis=0, keepdims=True) / N
    var = jnp.sum(y * y, axis=0, keepdims=True) / N - mu * mu
    y = gamma * (y - mu) / jnp.sqrt(var + 1e-5) + beta
    y = jnp.maximum(y, 0.0) * mask
    o = jnp.dot(y, W2, preferred_element_type=jnp.float32,
                precision=lax.Precision.HIGHEST) + b2
    return jnp.maximum(o, 0.0) * mask


def _layer_math(h, agg, W1, b1, gamma, beta, W2, b2):
    mask = (lax.broadcasted_iota(jnp.int32, (NP, 1), 0) < N).astype(
        jnp.float32)
    z = h + agg
    y = jnp.dot(z, W1, preferred_element_type=jnp.float32,
                precision=lax.Precision.HIGHEST) + b1
    return _bn_relu_mm(y, gamma, beta, W2, b2, mask)


def _dense0_body(u_ref, a_ref, b1_ref, g_ref, be_ref, W2_ref, b2_ref, o_ref):
    mask = (lax.broadcasted_iota(jnp.int32, (NP, 1), 0) < N).astype(
        jnp.float32)
    y = u_ref[...] + a_ref[...] + b1_ref[...]
    o_ref[...] = _bn_relu_mm(y, g_ref[...], be_ref[...], W2_ref[...],
                             b2_ref[...], mask)


def _dense0(u, agg, b1, gamma, beta, W2, b2):
    return pl.pallas_call(
        _dense0_body,
        out_shape=jax.ShapeDtypeStruct((NP, H), jnp.float32),
    )(u, agg, b1.reshape(1, H), gamma.reshape(1, H), beta.reshape(1, H),
      W2, b2.reshape(1, H))


def _dense_body(h_ref, a_ref, W1_ref, b1_ref, g_ref, be_ref, W2_ref, b2_ref,
                o_ref):
    o_ref[...] = _layer_math(h_ref[...], a_ref[...], W1_ref[...],
                             b1_ref[...], g_ref[...], be_ref[...],
                             W2_ref[...], b2_ref[...])


def _dense(h, agg, W1, b1, gamma, beta, W2, b2):
    return pl.pallas_call(
        _dense_body,
        out_shape=jax.ShapeDtypeStruct((NP, H), jnp.float32),
    )(h, agg, W1, b1.reshape(1, H), gamma.reshape(1, H), beta.reshape(1, H),
      W2, b2.reshape(1, H))


def _final_body(h_ref, a_ref, W1_ref, b1_ref, g_ref, be_ref, W2_ref, b2_ref,
                batch_ref, Wm1_ref, bm1_ref, Wm2_ref, bm2_ref, o_ref):
    h3 = _layer_math(h_ref[...], a_ref[...], W1_ref[...],
                     b1_ref[...], g_ref[...], be_ref[...], W2_ref[...],
                     b2_ref[...])
    onehot = (batch_ref[...] ==
              lax.broadcasted_iota(jnp.int32, (NP, NGRAPHS), 1)
              ).astype(jnp.float32)
    pooled = lax.dot_general(onehot, h3, (((0,), (0,)), ((), ())),
                             preferred_element_type=jnp.float32,
                             precision=lax.Precision.HIGHEST)
    y = jnp.maximum(
        jnp.dot(pooled, Wm1_ref[...], preferred_element_type=jnp.float32,
                precision=lax.Precision.HIGHEST) + bm1_ref[...], 0.0)
    o_ref[...] = jnp.dot(y, Wm2_ref[...], preferred_element_type=jnp.float32,
                         precision=lax.Precision.HIGHEST) + bm2_ref[...]


def _final(h, agg, W1, b1, gamma, beta, W2, b2, batch_p, Wm1, bm1, Wm2, bm2):
    return pl.pallas_call(
        _final_body,
        out_shape=jax.ShapeDtypeStruct((NGRAPHS, OUT), jnp.float32),
    )(h, agg, W1, b1.reshape(1, H), gamma.reshape(1, H), beta.reshape(1, H),
      W2, b2.reshape(1, H), batch_p, Wm1, bm1.reshape(1, H), Wm2,
      bm2.reshape(1, OUT))


def kernel(x, edge_index, edge_attr, batch, group_emb, hemi_emb,
           W1_0, b1_0, gamma_0, beta_0, W2_0, b2_0,
           W1_1, b1_1, gamma_1, beta_1, W2_1, b2_1,
           W1_2, b1_2, gamma_2, beta_2, W2_2, b2_2,
           Wm1, bm1, Wm2, bm2):
    src = edge_index[0]
    dst = edge_index[1]
    pad_e = EPAD - E
    ep = jnp.concatenate([
        jnp.bitwise_or(jnp.left_shift(src, PK), dst),
        jnp.full((pad_e,), PKM, jnp.int32),
    ])
    x_p = jnp.pad(x, ((0, NP - N), (0, 0)))
    batch_p = jnp.pad(batch, (0, NP - N),
                      constant_values=NGRAPHS).reshape(NP, 1)

    u0 = _embed(x_p, group_emb, hemi_emb, W1_0[:D], W1_0[D:])
    agg0 = _seg_sum(u0, ep)
    h1 = _dense0(u0, agg0, b1_0, gamma_0, beta_0, W2_0, b2_0)
    agg1 = _seg_sum(h1, ep)
    h2 = _dense(h1, agg1, W1_1, b1_1, gamma_1, beta_1, W2_1, b2_1)
    agg2 = _seg_sum(h2, ep)
    return _final(h2, agg2, W1_2, b1_2, gamma_2, beta_2, W2_2, b2_2,
                  batch_p, Wm1, bm1, Wm2, bm2)
